# baseline (speedup 1.0000x reference)
import functools

import jax
import jax.numpy as jnp
from jax import lax
from jax.experimental import pallas as pl
from jax.experimental.pallas import tpu as pltpu
from jax.experimental.pallas import tpu_sc as plsc

D = 16

_info = plsc.get_sparse_core_info()
_NC, _NS, _L = _info.num_cores, _info.num_subcores, _info.num_lanes
_NW = _NC * _NS

_BN = 65536


def _tc_matvec_body(matT_ref, w_ref, b_ref, sig_ref):
    s = jax.lax.dot_general(
        w_ref[...], matT_ref[...],
        (((1,), (0,)), ((), ())),
        preferred_element_type=jnp.float32,
    )
    z = s[0] + b_ref[0, 0]
    sig_ref[...] = 1.0 / (1.0 + jnp.exp(-z))


@functools.lru_cache(maxsize=None)
def _make_tc_matvec(nrows: int):
    grid = (nrows + _BN - 1) // _BN
    return pl.pallas_call(
        _tc_matvec_body,
        grid=(grid,),
        in_specs=[
            pl.BlockSpec((D, _BN), lambda i: (0, i)),
            pl.BlockSpec((1, D), lambda i: (0, 0)),
            pl.BlockSpec((1, 1), lambda i: (0, 0)),
        ],
        out_specs=pl.BlockSpec((_BN,), lambda i: (i,)),
        out_shape=jax.ShapeDtypeStruct((nrows,), jnp.float32),
    )


@functools.lru_cache(maxsize=None)
def _make_sc_gather(batch: int):
    bpw = batch // _NW
    chunk = min(128, bpw)
    nchunk = bpw // chunk
    groups_per_chunk = chunk // _L
    mesh = plsc.VectorSubcoreMesh(core_axis_name="c", subcore_axis_name="s")

    @functools.partial(
        pl.kernel,
        out_type=jax.ShapeDtypeStruct((batch,), jnp.float32),
        mesh=mesh,
        compiler_params=pltpu.CompilerParams(
            use_tc_tiling_on_sc=False, needs_layout_passes=False),
        scratch_types=[
            pltpu.VMEM((bpw,), jnp.int32),
            pltpu.VMEM((bpw,), jnp.float32),
            pltpu.SemaphoreType.DMA,
        ],
    )
    def sc_gather(x_hbm, s_hbm, out_hbm, idx_v, sv_v, sem):
        wid = lax.axis_index("s") * _NC + lax.axis_index("c")
        base = wid * bpw
        pltpu.sync_copy(x_hbm.at[pl.ds(base, bpw)], idx_v)

        copies = [
            pltpu.async_copy(
                s_hbm.at[idx_v.at[pl.ds(c * chunk, chunk)]],
                sv_v.at[pl.ds(c * chunk, chunk)],
                sem,
            )
            for c in range(nchunk)
        ]
        for cp in copies:
            cp.wait()

        pltpu.sync_copy(sv_v, out_hbm.at[pl.ds(base, bpw)])

    return sc_gather


def kernel(x, mat, w, b):
    batch = x.shape[0]
    xi = x.astype(jnp.int32)
    wf = w.reshape(1, D).astype(jnp.float32)
    bf = b.astype(jnp.float32).reshape(1, 1)
    sig = _make_tc_matvec(mat.shape[0])(mat.T, wf, bf)
    out = _make_sc_gather(batch)(xi, sig)
    return out.reshape(batch, 1)


# speedup vs baseline: 1.0449x; 1.0449x over previous
import functools

import jax
import jax.numpy as jnp
from jax import lax
from jax.experimental import pallas as pl
from jax.experimental.pallas import tpu as pltpu
from jax.experimental.pallas import tpu_sc as plsc

D = 16

_info = plsc.get_sparse_core_info()
_NC, _NS, _L = _info.num_cores, _info.num_subcores, _info.num_lanes
_NW = _NC * _NS

_BN = 196608


def _tc_matvec_body(matT_ref, w_ref, b_ref, sig_ref):
    s = jax.lax.dot_general(
        w_ref[...], matT_ref[...],
        (((1,), (0,)), ((), ())),
        preferred_element_type=jnp.float32,
    )
    z = s[0] + b_ref[0, 0]
    sig_ref[...] = 1.0 / (1.0 + jnp.exp(-z))


@functools.lru_cache(maxsize=None)
def _make_tc_matvec(nrows: int):
    grid = (nrows + _BN - 1) // _BN
    return pl.pallas_call(
        _tc_matvec_body,
        grid=(grid,),
        in_specs=[
            pl.BlockSpec((D, _BN), lambda i: (0, i)),
            pl.BlockSpec((1, D), lambda i: (0, 0)),
            pl.BlockSpec((1, 1), lambda i: (0, 0)),
        ],
        out_specs=pl.BlockSpec((_BN,), lambda i: (i,)),
        out_shape=jax.ShapeDtypeStruct((nrows,), jnp.float32),
    )


@functools.lru_cache(maxsize=None)
def _make_sc_gather(batch: int):
    bpw = batch // _NW
    chunk = min(128, bpw)
    nchunk = bpw // chunk
    groups_per_chunk = chunk // _L
    mesh = plsc.VectorSubcoreMesh(core_axis_name="c", subcore_axis_name="s")

    @functools.partial(
        pl.kernel,
        out_type=jax.ShapeDtypeStruct((batch,), jnp.float32),
        mesh=mesh,
        compiler_params=pltpu.CompilerParams(
            use_tc_tiling_on_sc=False, needs_layout_passes=False),
        scratch_types=[
            pltpu.VMEM((bpw,), jnp.int32),
            pltpu.VMEM((bpw,), jnp.float32),
            pltpu.SemaphoreType.DMA,
        ],
    )
    def sc_gather(x_hbm, s_hbm, out_hbm, idx_v, sv_v, sem):
        wid = lax.axis_index("s") * _NC + lax.axis_index("c")
        base = wid * bpw
        pltpu.sync_copy(x_hbm.at[pl.ds(base, bpw)], idx_v)

        copies = [
            pltpu.async_copy(
                s_hbm.at[idx_v.at[pl.ds(c * chunk, chunk)]],
                sv_v.at[pl.ds(c * chunk, chunk)],
                sem,
            )
            for c in range(nchunk)
        ]
        for cp in copies:
            cp.wait()

        pltpu.sync_copy(sv_v, out_hbm.at[pl.ds(base, bpw)])

    return sc_gather


def kernel(x, mat, w, b):
    batch = x.shape[0]
    xi = x.astype(jnp.int32)
    wf = w.reshape(1, D).astype(jnp.float32)
    bf = b.astype(jnp.float32).reshape(1, 1)
    sig = _make_tc_matvec(mat.shape[0])(mat.T, wf, bf)
    out = _make_sc_gather(batch)(xi, sig)
    return out.reshape(batch, 1)


# speedup vs baseline: 1.0880x; 1.0412x over previous
import functools

import jax
import jax.numpy as jnp
from jax import lax
from jax.experimental import pallas as pl
from jax.experimental.pallas import tpu as pltpu
from jax.experimental.pallas import tpu_sc as plsc

D = 16

_info = plsc.get_sparse_core_info()
_NC, _NS, _L = _info.num_cores, _info.num_subcores, _info.num_lanes
_NW = _NC * _NS

_BN = 131072


def _tc_matvec_body(matT_ref, w_ref, b_ref, sig_ref):
    s = jax.lax.dot_general(
        w_ref[...], matT_ref[...],
        (((1,), (0,)), ((), ())),
        preferred_element_type=jnp.float32,
    )
    z = s[0] + b_ref[0, 0]
    sig_ref[...] = 1.0 / (1.0 + jnp.exp(-z))


@functools.lru_cache(maxsize=None)
def _make_tc_matvec(nrows: int):
    grid = (nrows + _BN - 1) // _BN
    return pl.pallas_call(
        _tc_matvec_body,
        grid=(grid,),
        in_specs=[
            pl.BlockSpec((D, _BN), lambda i: (0, i)),
            pl.BlockSpec((1, D), lambda i: (0, 0)),
            pl.BlockSpec((1, 1), lambda i: (0, 0)),
        ],
        out_specs=pl.BlockSpec((_BN,), lambda i: (i,)),
        out_shape=jax.ShapeDtypeStruct((nrows,), jnp.float32),
    )


@functools.lru_cache(maxsize=None)
def _make_sc_gather(batch: int):
    bpw = batch // _NW
    chunk = min(128, bpw)
    nchunk = bpw // chunk
    groups_per_chunk = chunk // _L
    mesh = plsc.VectorSubcoreMesh(core_axis_name="c", subcore_axis_name="s")

    @functools.partial(
        pl.kernel,
        out_type=jax.ShapeDtypeStruct((batch,), jnp.float32),
        mesh=mesh,
        compiler_params=pltpu.CompilerParams(
            use_tc_tiling_on_sc=False, needs_layout_passes=False),
        scratch_types=[
            pltpu.VMEM((bpw,), jnp.int32),
            pltpu.VMEM((bpw,), jnp.float32),
            pltpu.SemaphoreType.DMA,
        ],
    )
    def sc_gather(x_hbm, s_hbm, out_hbm, idx_v, sv_v, sem):
        wid = lax.axis_index("s") * _NC + lax.axis_index("c")
        base = wid * bpw
        pltpu.sync_copy(x_hbm.at[pl.ds(base, bpw)], idx_v)

        copies = [
            pltpu.async_copy(
                s_hbm.at[idx_v.at[pl.ds(c * chunk, chunk)]],
                sv_v.at[pl.ds(c * chunk, chunk)],
                sem,
            )
            for c in range(nchunk)
        ]
        for cp in copies:
            cp.wait()

        pltpu.sync_copy(sv_v, out_hbm.at[pl.ds(base, bpw)])

    return sc_gather


def kernel(x, mat, w, b):
    batch = x.shape[0]
    xi = x.astype(jnp.int32)
    wf = w.reshape(1, D).astype(jnp.float32)
    bf = b.astype(jnp.float32).reshape(1, 1)
    sig = _make_tc_matvec(mat.shape[0])(mat.T, wf, bf)
    out = _make_sc_gather(batch)(xi, sig)
    return out.reshape(batch, 1)
